# baseline (speedup 1.0000x reference)
import jax
import jax.numpy as jnp
from jax import lax
from jax.experimental import pallas as pl
from jax.experimental.pallas import tpu as pltpu
from jax.experimental.pallas import tpu_sc as plsc

B = 16384
DIM = 64
NC = 2
NS = 16
NW = NC * NS
BPW = B // NW
CHUNK = 128
NCHUNK = BPW // CHUNK


def _score_chunk(s_v, r_v, t_v, o_v, out_v, out_base):
    lane = lax.iota(jnp.int32, 16)

    def group_body(g, carry):
        ir = g * 16 + lane

        def dim_body(dc, acc):
            for j in range(8):
                d = dc * 8 + j
                ic = jnp.full((16,), d, jnp.int32)
                sv = plsc.load_gather(s_v, [ir, ic])
                rv = plsc.load_gather(r_v, [ir, ic])
                tv = plsc.load_gather(t_v, [ir, ic])
                ov = plsc.load_gather(o_v, [ir, ic])
                acc = acc + jnp.abs(sv + rv + tv - ov)
            return acc

        acc = lax.fori_loop(0, 8, dim_body, jnp.zeros((16,), jnp.float32))
        out_v[pl.ds(out_base + g * 16, 16)] = acc
        return carry

    lax.fori_loop(0, 8, group_body, 0)


def _body(pos_s, pos_r, pos_o, pos_t, neg_s, neg_r, neg_o, neg_t,
          e_w, r_w, t_w, pos_out, neg_out,
          ps_v, pr_v, po_v, pt_v, ns_v, nr_v, no_v,
          s_v, r_v, t_v, o_v, pos_ov, neg_ov, sem):
    wid = lax.axis_index("s") * NC + lax.axis_index("c")
    base = wid * BPW

    pltpu.sync_copy(pos_s.at[pl.ds(base, BPW)], ps_v)
    pltpu.sync_copy(pos_r.at[pl.ds(base, BPW)], pr_v)
    pltpu.sync_copy(pos_o.at[pl.ds(base, BPW)], po_v)
    pltpu.sync_copy(pos_t.at[pl.ds(base, BPW)], pt_v)
    pltpu.sync_copy(neg_s.at[pl.ds(base, BPW)], ns_v)
    pltpu.sync_copy(neg_r.at[pl.ds(base, BPW)], nr_v)
    pltpu.sync_copy(neg_o.at[pl.ds(base, BPW)], no_v)

    for c in range(NCHUNK):
        cb = c * CHUNK
        cps = pltpu.async_copy(e_w.at[ps_v.at[pl.ds(cb, CHUNK)]], s_v, sem)
        cpr = pltpu.async_copy(r_w.at[pr_v.at[pl.ds(cb, CHUNK)]], r_v, sem)
        cpt = pltpu.async_copy(t_w.at[pt_v.at[pl.ds(cb, CHUNK)]], t_v, sem)
        cpo = pltpu.async_copy(e_w.at[po_v.at[pl.ds(cb, CHUNK)]], o_v, sem)
        cps.wait(); cpr.wait(); cpt.wait(); cpo.wait()
        _score_chunk(s_v, r_v, t_v, o_v, pos_ov, cb)

        cns = pltpu.async_copy(e_w.at[ns_v.at[pl.ds(cb, CHUNK)]], s_v, sem)
        cnr = pltpu.async_copy(r_w.at[nr_v.at[pl.ds(cb, CHUNK)]], r_v, sem)
        cno = pltpu.async_copy(e_w.at[no_v.at[pl.ds(cb, CHUNK)]], o_v, sem)
        cns.wait(); cnr.wait(); cno.wait()
        _score_chunk(s_v, r_v, t_v, o_v, neg_ov, cb)

    pltpu.sync_copy(pos_ov, pos_out.at[pl.ds(base, BPW)])
    pltpu.sync_copy(neg_ov, neg_out.at[pl.ds(base, BPW)])


def kernel(pos_s, pos_r, pos_o, pos_t, neg_s, neg_r, neg_o, neg_t,
           e_weight, r_weight, t_weight):
    mesh = plsc.VectorSubcoreMesh(
        core_axis_name="c", subcore_axis_name="s",
        num_cores=NC, num_subcores=NS)
    f32 = jnp.float32
    run = pl.kernel(
        _body,
        out_type=(jax.ShapeDtypeStruct((B,), f32),
                  jax.ShapeDtypeStruct((B,), f32)),
        mesh=mesh,
        scratch_types=[
            pltpu.VMEM((BPW,), jnp.int32),
            pltpu.VMEM((BPW,), jnp.int32),
            pltpu.VMEM((BPW,), jnp.int32),
            pltpu.VMEM((BPW,), jnp.int32),
            pltpu.VMEM((BPW,), jnp.int32),
            pltpu.VMEM((BPW,), jnp.int32),
            pltpu.VMEM((BPW,), jnp.int32),
            pltpu.VMEM((CHUNK, DIM), f32),
            pltpu.VMEM((CHUNK, DIM), f32),
            pltpu.VMEM((CHUNK, DIM), f32),
            pltpu.VMEM((CHUNK, DIM), f32),
            pltpu.VMEM((BPW,), f32),
            pltpu.VMEM((BPW,), f32),
            pltpu.SemaphoreType.DMA,
        ],
        compiler_params=pltpu.CompilerParams(
            needs_layout_passes=False, use_tc_tiling_on_sc=False),
    )
    return run(pos_s.astype(jnp.int32), pos_r.astype(jnp.int32),
               pos_o.astype(jnp.int32), pos_t.astype(jnp.int32),
               neg_s.astype(jnp.int32), neg_r.astype(jnp.int32),
               neg_o.astype(jnp.int32), neg_t.astype(jnp.int32),
               e_weight, r_weight, t_weight)


# speedup vs baseline: 1.1603x; 1.1603x over previous
import jax
import jax.numpy as jnp
from jax import lax
from jax.experimental import pallas as pl
from jax.experimental.pallas import tpu as pltpu
from jax.experimental.pallas import tpu_sc as plsc

B = 16384
DIM = 64
NC = 2
NS = 16
NW = NC * NS
BPW = B // NW
CHUNK = 128
NCHUNK = BPW // CHUNK
NTASK = 2 * NCHUNK


def _score_chunk(s_v, r_v, t_v, o_v, out_v, out_base):
    lane = lax.iota(jnp.int32, 16)
    last = lane == 15

    def elem(i, carry):
        total = jnp.zeros((16,), jnp.float32)
        for q in range(DIM // 16):
            sl = pl.ds(q * 16, 16)
            total = total + jnp.abs(
                s_v[i, sl] + r_v[i, sl] + t_v[i, sl] - o_v[i, sl])
        csum = plsc.cumsum(total)
        idx = jnp.full((16,), out_base + i, jnp.int32)
        plsc.store_scatter(out_v, [idx], csum, mask=last)
        return carry

    lax.fori_loop(0, CHUNK, elem, 0)


def _body(pos_s, pos_r, pos_o, pos_t, neg_s, neg_r, neg_o, neg_t,
          e_w, r_w, t_w, pos_out, neg_out,
          ps_v, pr_v, po_v, pt_v, ns_v, nr_v, no_v,
          s_v0, r_v0, t_v0, o_v0, s_v1, r_v1, t_v1, o_v1,
          pos_ov, neg_ov, sem0, sem1):
    wid = lax.axis_index("s") * NC + lax.axis_index("c")
    base = wid * BPW

    pltpu.sync_copy(pos_s.at[pl.ds(base, BPW)], ps_v)
    pltpu.sync_copy(pos_r.at[pl.ds(base, BPW)], pr_v)
    pltpu.sync_copy(pos_o.at[pl.ds(base, BPW)], po_v)
    pltpu.sync_copy(pos_t.at[pl.ds(base, BPW)], pt_v)
    pltpu.sync_copy(neg_s.at[pl.ds(base, BPW)], ns_v)
    pltpu.sync_copy(neg_r.at[pl.ds(base, BPW)], nr_v)
    pltpu.sync_copy(neg_o.at[pl.ds(base, BPW)], no_v)

    bufs = ((s_v0, r_v0, t_v0, o_v0), (s_v1, r_v1, t_v1, o_v1))
    sems = (sem0, sem1)
    tasks = [(c, (ps_v, pr_v, pt_v, po_v), pos_ov) for c in range(NCHUNK)]
    tasks += [(c, (ns_v, nr_v, pt_v, no_v), neg_ov) for c in range(NCHUNK)]

    def fire(k):
        c, (si, ri, ti, oi), _ = tasks[k]
        sb, rb, tb, ob = bufs[k % 2]
        sem = sems[k % 2]
        cb = c * CHUNK
        return (pltpu.async_copy(e_w.at[si.at[pl.ds(cb, CHUNK)]], sb, sem),
                pltpu.async_copy(r_w.at[ri.at[pl.ds(cb, CHUNK)]], rb, sem),
                pltpu.async_copy(t_w.at[ti.at[pl.ds(cb, CHUNK)]], tb, sem),
                pltpu.async_copy(e_w.at[oi.at[pl.ds(cb, CHUNK)]], ob, sem))

    pending = fire(0)
    for k in range(NTASK):
        for cp in pending:
            cp.wait()
        if k + 1 < NTASK:
            nxt = fire(k + 1)
        c, _, out_v = tasks[k]
        sb, rb, tb, ob = bufs[k % 2]
        _score_chunk(sb, rb, tb, ob, out_v, c * CHUNK)
        if k + 1 < NTASK:
            pending = nxt

    pltpu.sync_copy(pos_ov, pos_out.at[pl.ds(base, BPW)])
    pltpu.sync_copy(neg_ov, neg_out.at[pl.ds(base, BPW)])


def kernel(pos_s, pos_r, pos_o, pos_t, neg_s, neg_r, neg_o, neg_t,
           e_weight, r_weight, t_weight):
    mesh = plsc.VectorSubcoreMesh(
        core_axis_name="c", subcore_axis_name="s",
        num_cores=NC, num_subcores=NS)
    f32 = jnp.float32
    run = pl.kernel(
        _body,
        out_type=(jax.ShapeDtypeStruct((B,), f32),
                  jax.ShapeDtypeStruct((B,), f32)),
        mesh=mesh,
        scratch_types=[
            pltpu.VMEM((BPW,), jnp.int32),
            pltpu.VMEM((BPW,), jnp.int32),
            pltpu.VMEM((BPW,), jnp.int32),
            pltpu.VMEM((BPW,), jnp.int32),
            pltpu.VMEM((BPW,), jnp.int32),
            pltpu.VMEM((BPW,), jnp.int32),
            pltpu.VMEM((BPW,), jnp.int32),
            pltpu.VMEM((CHUNK, DIM), f32),
            pltpu.VMEM((CHUNK, DIM), f32),
            pltpu.VMEM((CHUNK, DIM), f32),
            pltpu.VMEM((CHUNK, DIM), f32),
            pltpu.VMEM((CHUNK, DIM), f32),
            pltpu.VMEM((CHUNK, DIM), f32),
            pltpu.VMEM((CHUNK, DIM), f32),
            pltpu.VMEM((CHUNK, DIM), f32),
            pltpu.VMEM((BPW,), f32),
            pltpu.VMEM((BPW,), f32),
            pltpu.SemaphoreType.DMA,
            pltpu.SemaphoreType.DMA,
        ],
        compiler_params=pltpu.CompilerParams(
            needs_layout_passes=False, use_tc_tiling_on_sc=False),
    )
    return run(pos_s.astype(jnp.int32), pos_r.astype(jnp.int32),
               pos_o.astype(jnp.int32), pos_t.astype(jnp.int32),
               neg_s.astype(jnp.int32), neg_r.astype(jnp.int32),
               neg_o.astype(jnp.int32), neg_t.astype(jnp.int32),
               e_weight, r_weight, t_weight)
